# baseline (speedup 1.0000x reference)
import functools

import jax
import jax.numpy as jnp
from jax.experimental import pallas as pl
from jax.experimental.pallas import tpu as pltpu


def _loss_body(n_total, nc, p_ref, t_ref, rp_ref, rt_ref, cnt_ref, thres_ref,
               out_ref, acc_ref):
    b = pl.program_id(0)
    c = pl.program_id(1)
    num_c = pl.num_programs(1)

    @pl.when(jnp.logical_and(b == 0, c == 0))
    def _():
        out_ref[...] = jnp.zeros((1, 1), jnp.float32)

    @pl.when(c == 0)
    def _():
        acc_ref[0] = jnp.float32(0.0)
        acc_ref[1] = jnp.float32(0.0)

    p = p_ref[...]
    px1 = p[:, 0:1]
    py1 = p[:, 1:2]
    px2 = p[:, 2:3]
    py2 = p[:, 3:4]
    t = t_ref[...]
    tx1 = t[0:1, :]
    ty1 = t[1:2, :]
    tx2 = t[2:3, :]
    ty2 = t[3:4, :]

    zero = jnp.float32(0.0)
    area_p = jnp.maximum(px2 - px1, zero) * jnp.maximum(py2 - py1, zero)
    area_t = jnp.maximum(tx2 - tx1, zero) * jnp.maximum(ty2 - ty1, zero)

    w = jnp.maximum(jnp.minimum(px2, tx2) - jnp.maximum(px1, tx1), zero)
    h = jnp.maximum(jnp.minimum(py2, ty2) - jnp.maximum(py1, ty1), zero)
    inter = w * h
    union = area_p + area_t - inter
    thres = thres_ref[0]
    hit = inter > thres * jnp.maximum(union, jnp.float32(1e-9))
    anyhit = jnp.any(hit, axis=1, keepdims=True)

    cnt = cnt_ref[...]
    row = c * nc + jax.lax.broadcasted_iota(jnp.int32, (nc, 1), 0)
    valid = row < n_total
    pos = jnp.logical_and(jnp.logical_and(anyhit, cnt > zero), valid)

    d = rp_ref[...] - rt_ref[...]
    ad = jnp.abs(d)
    sl1 = jnp.where(ad < jnp.float32(1.0), jnp.float32(0.5) * d * d,
                    ad - jnp.float32(0.5))
    rowsum = jnp.sum(sl1, axis=1, keepdims=True)
    contrib = jnp.where(pos, rowsum, zero)

    acc_ref[0] += jnp.sum(pos.astype(jnp.float32))
    acc_ref[1] += jnp.sum(contrib)

    @pl.when(c == num_c - 1)
    def _():
        npos = acc_ref[0]
        lsum = acc_ref[1]
        img = jnp.where(npos > zero, lsum / npos, zero)
        nb = pl.num_programs(0)
        out_ref[...] = out_ref[...] + (img / jnp.float32(nb)).reshape(1, 1)


def kernel(P_bbx, cls_logits, reg_preds, T_boxes, cnt_p57, reg_p57, cnt_p2,
           reg_p2, iou_thres):
    del cls_logits
    B, N, _ = P_bbx.shape
    reg_t = jnp.concatenate([reg_p2.reshape(B, -1, 5), reg_p57], axis=1)
    cnt_t = jnp.concatenate([cnt_p2.reshape(B, -1, 1), cnt_p57], axis=1)
    Tt = jnp.transpose(T_boxes, (0, 2, 1))
    thres = jnp.reshape(iou_thres, (1,)).astype(jnp.float32)

    NC = 2048
    num_c = (N + NC - 1) // NC

    grid = (B, num_c)
    out = pl.pallas_call(
        functools.partial(_loss_body, N, NC),
        grid=grid,
        in_specs=[
            pl.BlockSpec((None, NC, 4), lambda b, c: (b, c, 0)),
            pl.BlockSpec((None, 4, T_boxes.shape[1]), lambda b, c: (b, 0, 0)),
            pl.BlockSpec((None, NC, 5), lambda b, c: (b, c, 0)),
            pl.BlockSpec((None, NC, 5), lambda b, c: (b, c, 0)),
            pl.BlockSpec((None, NC, 1), lambda b, c: (b, c, 0)),
            pl.BlockSpec(memory_space=pltpu.SMEM),
        ],
        out_specs=pl.BlockSpec((1, 1), lambda b, c: (0, 0)),
        out_shape=jax.ShapeDtypeStruct((1, 1), jnp.float32),
        scratch_shapes=[pltpu.SMEM((2,), jnp.float32)],
    )(P_bbx, Tt, reg_preds, reg_t, cnt_t, thres)
    return out


# speedup vs baseline: 15.1296x; 15.1296x over previous
import functools

import jax
import jax.numpy as jnp
from jax.experimental import pallas as pl
from jax.experimental.pallas import tpu as pltpu

_LANES = 128
_ROWS = 136
_M = 32


def _loss_body(c_ref, r_ref, cnt_ref, t_ref, thres_ref, out_ref):
    b = pl.program_id(0)
    nb = pl.num_programs(0)

    @pl.when(b == 0)
    def _():
        out_ref[...] = jnp.zeros((1, 1), jnp.float32)

    zero = jnp.float32(0.0)
    thres = thres_ref[0]
    c1 = jnp.float32(1.0) + thres

    tx1 = [t_ref[b, m, 0] for m in range(_M)]
    ty1 = [t_ref[b, m, 1] for m in range(_M)]
    tx2 = [t_ref[b, m, 2] for m in range(_M)]
    ty2 = [t_ref[b, m, 3] for m in range(_M)]
    atm = [thres * (jnp.maximum(tx2[m] - tx1[m], zero)
                    * jnp.maximum(ty2[m] - ty1[m], zero)) for m in range(_M)]

    npos_acc = jnp.zeros((8, _LANES), jnp.float32)
    loss_acc = jnp.zeros((8, _LANES), jnp.float32)

    for r0, rows in ((0, 72), (72, 64)):
        sl = pl.ds(r0, rows)
        px1 = c_ref[0, sl, :]
        py1 = c_ref[1, sl, :]
        px2 = c_ref[2, sl, :]
        py2 = c_ref[3, sl, :]
        area_p = (jnp.maximum(px2 - px1, zero)
                  * jnp.maximum(py2 - py1, zero))
        apt = thres * area_p

        hit = jnp.zeros((rows, _LANES), jnp.bool_)
        for m in range(_M):
            w = jnp.maximum(
                jnp.minimum(px2, tx2[m]) - jnp.maximum(px1, tx1[m]), zero)
            h = jnp.minimum(py2, ty2[m]) - jnp.maximum(py1, ty1[m])
            inter = w * h
            hit = jnp.logical_or(hit, inter * c1 > apt + atm[m])

        pos = jnp.where(jnp.logical_and(hit, cnt_ref[sl, :] > zero),
                        jnp.float32(1.0), zero)

        rowsum = jnp.zeros((rows, _LANES), jnp.float32)
        for k in range(5):
            d = r_ref[k, sl, :] - r_ref[5 + k, sl, :]
            ad = jnp.abs(d)
            rowsum = rowsum + jnp.where(
                ad < jnp.float32(1.0),
                jnp.float32(0.5) * d * d, ad - jnp.float32(0.5))

        for v in range(rows // 8):
            npos_acc = npos_acc + pos[8 * v:8 * v + 8, :]
            loss_acc = loss_acc + (rowsum * pos)[8 * v:8 * v + 8, :]

    npos = jnp.sum(npos_acc)
    lsum = jnp.sum(loss_acc)
    img = jnp.where(npos > zero, lsum / npos, zero)
    out_ref[...] = out_ref[...] + (img / jnp.float32(nb)).reshape(1, 1)


def kernel(P_bbx, cls_logits, reg_preds, T_boxes, cnt_p57, reg_p57, cnt_p2,
           reg_p2, iou_thres):
    del cls_logits
    B, N, _ = P_bbx.shape
    npad = _ROWS * _LANES

    pads = ((0, 0), (0, npad - N), (0, 0))
    coords = jnp.pad(P_bbx, pads).transpose(0, 2, 1).reshape(
        B, 4, _ROWS, _LANES)
    reg_t = jnp.concatenate([reg_p2.reshape(B, -1, 5), reg_p57], axis=1)
    regs = jnp.pad(jnp.concatenate([reg_preds, reg_t], axis=2),
                   pads).transpose(0, 2, 1).reshape(B, 10, _ROWS, _LANES)
    cnt = jnp.pad(
        jnp.concatenate([cnt_p2.reshape(B, -1), cnt_p57.reshape(B, -1)],
                        axis=1),
        ((0, 0), (0, npad - N))).reshape(B, _ROWS, _LANES)
    thres = jnp.reshape(iou_thres, (1,)).astype(jnp.float32)

    out = pl.pallas_call(
        _loss_body,
        grid=(B,),
        in_specs=[
            pl.BlockSpec((None, 4, _ROWS, _LANES), lambda b: (b, 0, 0, 0)),
            pl.BlockSpec((None, 10, _ROWS, _LANES), lambda b: (b, 0, 0, 0)),
            pl.BlockSpec((None, _ROWS, _LANES), lambda b: (b, 0, 0)),
            pl.BlockSpec(memory_space=pltpu.SMEM),
            pl.BlockSpec(memory_space=pltpu.SMEM),
        ],
        out_specs=pl.BlockSpec((1, 1), lambda b: (0, 0)),
        out_shape=jax.ShapeDtypeStruct((1, 1), jnp.float32),
    )(coords, regs, cnt, T_boxes, thres)
    return out
